# baseline (speedup 1.0000x reference)
import functools

import jax
import jax.numpy as jnp
from jax import lax
from jax.experimental import pallas as pl
from jax.experimental.pallas import tpu as pltpu
from jax.experimental.pallas import tpu_sc as plsc

R = 128
V = 1_000_000
L = 16
NW = 32
NG = 16
TPC = 31
CW = TPC * 128
NCHUNK = 126
HALF = NCHUNK * CW
EPI_COL = 2 * HALF
EPI_W = 128
EPI_ID = NCHUNK
BIG = 2**31 - 1
NEG = float("-inf")


U = 8
GRP = U * L


def _row_max(buf, s):
    init = tuple(jnp.full((L,), NEG, jnp.float32) for _ in range(U))

    def body(i, accs):
        base = i * GRP
        return tuple(
            jnp.maximum(accs[u], buf[s, pl.ds(base + u * L, L)])
            for u in range(U)
        )

    accs = lax.fori_loop(0, CW // GRP, body, init)
    m01 = jnp.maximum(accs[0], accs[1])
    m23 = jnp.maximum(accs[2], accs[3])
    m45 = jnp.maximum(accs[4], accs[5])
    m67 = jnp.maximum(accs[6], accs[7])
    return jnp.maximum(jnp.maximum(m01, m23), jnp.maximum(m45, m67))


def _chunk_maxes(buf, _unused):
    return tuple(_row_max(buf, s) for s in range(8))


def _sc_argmax_body(x_hbm, tail_hbm, out_f_hbm, out_i_hbm, buf0, buf1, bufe,
                    res_f, res_i, sem0, sem1, seme):
    cid = lax.axis_index("c")
    sid = lax.axis_index("s")
    wid = sid * 2 + cid
    g = wid // 2
    h = wid % 2
    row0 = g * 8
    colbase = h * HALF

    lane = lax.iota(jnp.int32, L)

    def chunk_copy(col, buf, sem):
        return pltpu.make_async_copy(
            x_hbm.at[
                pl.ds(pl.multiple_of(row0, 8), 8),
                pl.ds(pl.multiple_of(col, 128), CW),
            ],
            buf,
            sem,
        )

    def start_chunk(col, buf, sem):
        chunk_copy(col, buf, sem).start()

    def wait_chunk(col, buf, sem):
        chunk_copy(col, buf, sem).wait()

    def esrc():
        return tail_hbm.at[pl.ds(pl.multiple_of(row0, 8), 8), :]

    pltpu.make_async_copy(esrc(), bufe, seme).start()
    start_chunk(colbase, buf0, sem0)

    zero8f = tuple(jnp.full((L,), NEG, jnp.float32) for _ in range(8))

    def pair_body(p, carry):
        gmax, bc = carry
        c0 = 2 * p
        start_chunk(colbase + (c0 + 1) * CW, buf1, sem1)
        wait_chunk(colbase + c0 * CW, buf0, sem0)
        cm0 = _chunk_maxes(buf0, zero8f)

        @pl.when(c0 + 2 < NCHUNK)
        def _():
            start_chunk(colbase + (c0 + 2) * CW, buf0, sem0)

        better = tuple(cm0[s] > gmax[s] for s in range(8))
        bc = tuple(jnp.where(better[s], c0, bc[s]) for s in range(8))
        gmax = tuple(jnp.maximum(gmax[s], cm0[s]) for s in range(8))

        wait_chunk(colbase + (c0 + 1) * CW, buf1, sem1)
        cm1 = _chunk_maxes(buf1, zero8f)
        better = tuple(cm1[s] > gmax[s] for s in range(8))
        bc = tuple(jnp.where(better[s], c0 + 1, bc[s]) for s in range(8))
        gmax = tuple(jnp.maximum(gmax[s], cm1[s]) for s in range(8))
        return gmax, bc

    gmax, bc = lax.fori_loop(
        0, NCHUNK // 2, pair_body,
        (zero8f, tuple(jnp.zeros((L,), jnp.int32) for _ in range(8))),
    )

    pltpu.make_async_copy(esrc(), bufe, seme).wait()
    epi_gate = jnp.where(h == 1, jnp.float32(float("inf")), jnp.float32(NEG))
    for s in range(8):
        em = jnp.full((L,), NEG, jnp.float32)
        for k in range(EPI_W // L):
            em = jnp.maximum(em, bufe[s, pl.ds(k * L, L)])
        em = jnp.minimum(em, epi_gate)
        better = em > gmax[s]
        bc = tuple(
            jnp.where(better, EPI_ID, bc[t]) if t == s else bc[t]
            for t in range(8)
        )
        gmax = tuple(
            jnp.where(better, em, gmax[t]) if t == s else gmax[t]
            for t in range(8)
        )

    resf = jnp.zeros((L,), jnp.float32)
    resi = jnp.zeros((L,), jnp.int32)

    for s in range(8):
        rmax = jnp.float32(NEG)
        rbc = jnp.int32(BIG)
        for l in range(L):
            v = gmax[s][l]
            c = bc[s][l]
            take = (v > rmax) | ((v == rmax) & (c < rbc))
            rbc = jnp.where(take, c, rbc)
            rmax = jnp.where(take, v, rmax)

        safe_bc = jnp.minimum(rbc, NCHUNK - 1)
        start_chunk(colbase + safe_bc * CW, buf0, sem0)
        wait_chunk(colbase + safe_bc * CW, buf0, sem0)

        def find_body(i, best, s=s, rmax=rmax):
            v = buf0[s, pl.ds(i * L, L)]
            idx = i * L + lane
            return jnp.minimum(best, jnp.where(v == rmax, idx, BIG))

        bestv = lax.fori_loop(
            0, CW // L, find_body, jnp.full((L,), BIG, jnp.int32)
        )
        off = jnp.int32(BIG)
        for l in range(L):
            off = jnp.minimum(off, bestv[l])

        ebest = jnp.full((L,), BIG, jnp.int32)
        for k in range(EPI_W // L):
            v = bufe[s, pl.ds(k * L, L)]
            ebest = jnp.minimum(
                ebest, jnp.where(v == rmax, k * L + lane, BIG)
            )
        eoff = jnp.int32(BIG)
        for l in range(L):
            eoff = jnp.minimum(eoff, ebest[l])

        col = jnp.where(
            rbc == EPI_ID, EPI_COL + eoff, colbase + safe_bc * CW + off
        )
        resf = jnp.where(lane == s, rmax, resf)
        resi = jnp.where(lane == s, col, resi)

    res_f[...] = resf
    res_i[...] = resi
    pltpu.sync_copy(res_f, out_f_hbm.at[wid])
    pltpu.sync_copy(res_i, out_i_hbm.at[wid])


_sc_argmax = functools.partial(
    pl.kernel,
    out_type=(
        jax.ShapeDtypeStruct((NW, L), jnp.float32),
        jax.ShapeDtypeStruct((NW, L), jnp.int32),
    ),
    mesh=plsc.VectorSubcoreMesh(core_axis_name="c", subcore_axis_name="s"),
    scratch_types=[
        pltpu.VMEM((8, CW), jnp.float32),
        pltpu.VMEM((8, CW), jnp.float32),
        pltpu.VMEM((8, EPI_W), jnp.float32),
        pltpu.VMEM((L,), jnp.float32),
        pltpu.VMEM((L,), jnp.int32),
        pltpu.SemaphoreType.DMA,
        pltpu.SemaphoreType.DMA,
        pltpu.SemaphoreType.DMA,
    ],
)(_sc_argmax_body)


def kernel(logits):
    tail = jnp.pad(
        logits[:, EPI_COL:], ((0, 0), (0, EPI_W - (V - EPI_COL))),
        constant_values=NEG,
    )
    out_f, out_i = _sc_argmax(logits, tail)
    f = out_f.reshape(NG, 2, L)[:, :, :8]
    i = out_i.reshape(NG, 2, L)[:, :, :8]
    take_hi = f[:, 1, :] > f[:, 0, :]
    return jnp.where(take_hi, i[:, 1, :], i[:, 0, :]).reshape(R)
